# baseline (speedup 1.0000x reference)
import jax
import jax.numpy as jnp
from jax.experimental import pallas as pl
from jax.experimental.pallas import tpu as pltpu

_N = 20000
_TM = 2000


def _elu(v):
    return jnp.where(v > 0, v, jnp.exp(v) - 1.0)


def _mega_body(x, wgt1, wgt2, wgt3, wgt4, wlt1, wlt2, wlt3, wlt4,
               bg1, bg2, bg3, bg4,
               pred, u1, u2, u3, u4,
               s0, s1, s2, s3, c1, c2, c3, c4):
    inv_n = 1.0 / _N
    steps = _N // _TM
    bf = jnp.bfloat16
    f32 = jnp.float32

    for s in (s0, s1, s2, s3):
        s[...] = jnp.zeros_like(s)

    def row_specs(k, o):
        return dict(in_specs=[pl.BlockSpec((_TM, k), lambda i: (i, 0))],
                    out_specs=[pl.BlockSpec((_TM, o), lambda i: (i, 0))])

    def head_body(x_v, u1_v):
        xb = x_v[...]
        u = jnp.dot(xb.astype(bf), wgt1[...],
                    preferred_element_type=f32) + bg1[...]
        u1_v[...] = u.astype(bf)
        s0[...] += jnp.sum(xb, axis=0, keepdims=True)

    def mid_body(c_ref, wgt, bg, s_ref):
        def body(uin_v, uout_v):
            h = _elu(uin_v[...].astype(f32) - c_ref[...])
            u = jnp.dot(h.astype(bf), wgt[...],
                        preferred_element_type=f32) + bg[...]
            uout_v[...] = u.astype(uout_v.dtype)
            s_ref[...] += jnp.sum(h, axis=0, keepdims=True)
        return body

    def mean_row(s_ref, wlt):
        return jnp.dot((s_ref[...] * inv_n).astype(bf), wlt[...],
                       preferred_element_type=f32)

    pltpu.emit_pipeline(head_body, grid=(steps,),
                        **row_specs(1033, 1000))(x, u1)
    c1[...] = mean_row(s0, wlt1)

    pltpu.emit_pipeline(mid_body(c1, wgt2, bg2, s1), grid=(steps,),
                        **row_specs(1000, 600))(u1, u2)
    c2[...] = mean_row(s1, wlt2)

    pltpu.emit_pipeline(mid_body(c2, wgt3, bg3, s2), grid=(steps,),
                        **row_specs(600, 300))(u2, u3)
    c3[...] = mean_row(s2, wlt3)

    pltpu.emit_pipeline(mid_body(c3, wgt4, bg4, s3), grid=(steps,),
                        **row_specs(300, 1))(u3, u4)
    c4[...] = mean_row(s3, wlt4)

    def tail_body(u4_v, pred_v):
        pred_v[...] = _elu(u4_v[...] - c4[...])

    pltpu.emit_pipeline(tail_body, grid=(steps,),
                        **row_specs(1, 1))(u4, pred)


def kernel(x, Wg1, bg1, Wl1, Wg2, bg2, Wl2, Wg3, bg3, Wl3, Wg4, bg4, Wl4):
    bf = jnp.bfloat16
    f32 = jnp.float32
    wgt = [w.T.astype(bf) for w in (Wg1, Wg2, Wg3, Wg4)]
    wlt = [w.T.astype(bf) for w in (Wl1, Wl2, Wl3, Wl4)]
    bgs = [b.reshape(1, -1).astype(f32) for b in (bg1, bg2, bg3, bg4)]

    hbm = pl.BlockSpec(memory_space=pltpu.MemorySpace.HBM)
    vmem = pl.BlockSpec(memory_space=pltpu.MemorySpace.VMEM)

    outs = pl.pallas_call(
        _mega_body,
        in_specs=[hbm] + [vmem] * 12,
        out_specs=[hbm] * 5,
        out_shape=[
            jax.ShapeDtypeStruct((_N, 1), f32),
            jax.ShapeDtypeStruct((_N, 1000), bf),
            jax.ShapeDtypeStruct((_N, 600), bf),
            jax.ShapeDtypeStruct((_N, 300), bf),
            jax.ShapeDtypeStruct((_N, 1), f32),
        ],
        scratch_shapes=[
            pltpu.VMEM((1, 1033), f32),
            pltpu.VMEM((1, 1000), f32),
            pltpu.VMEM((1, 600), f32),
            pltpu.VMEM((1, 300), f32),
            pltpu.VMEM((1, 1000), f32),
            pltpu.VMEM((1, 600), f32),
            pltpu.VMEM((1, 300), f32),
            pltpu.VMEM((1, 1), f32),
        ],
    )(x, *wgt, *wlt, *bgs)
    return outs[0]
